# baseline (speedup 1.0000x reference)
import jax
import jax.numpy as jnp
from jax.experimental import pallas as pl
from jax.experimental.pallas import tpu as pltpu

N_ROWS = 100000
T_COLS = 256
N_TGT = 128
BLOCK = 4000


def _body(tgt_ref, stim_ref, o_ref):
    b = pl.program_id(0)
    base = b * BLOCK
    o_ref[...] = jnp.zeros_like(o_ref)

    def add_one(j, carry):
        t = tgt_ref[j]

        @pl.when(jnp.logical_and(t >= base, t < base + BLOCK))
        def _():
            o_ref[pl.ds(t - base, 1), :] += stim_ref[pl.ds(j, 1), :]

        return carry

    jax.lax.fori_loop(0, N_TGT, add_one, 0)


def kernel(stimuli, targets):
    tgt = targets.astype(jnp.int32)

    grid_spec = pltpu.PrefetchScalarGridSpec(
        num_scalar_prefetch=1,
        grid=(N_ROWS // BLOCK,),
        in_specs=[
            pl.BlockSpec((N_TGT, T_COLS), lambda b, *_: (0, 0)),
        ],
        out_specs=pl.BlockSpec((BLOCK, T_COLS), lambda b, *_: (b, 0)),
    )
    return pl.pallas_call(
        _body,
        grid_spec=grid_spec,
        out_shape=jax.ShapeDtypeStruct((N_ROWS, T_COLS), jnp.float32),
    )(tgt, stimuli)


# speedup vs baseline: 1.5904x; 1.5904x over previous
import jax
import jax.numpy as jnp
from jax.experimental import pallas as pl
from jax.experimental.pallas import tpu as pltpu

N_ROWS = 100000
T_COLS = 256
N_TGT = 128
BLOCK = 4000


def _body(tgt_ref, rank_ref, starts_ref, stim_ref, o_ref, inv_ref):
    b = pl.program_id(0)

    @pl.when(b == 0)
    def _():
        def build(j, carry):
            inv_ref[rank_ref[j]] = j
            return carry

        jax.lax.fori_loop(0, N_TGT, build, 0)

    o_ref[...] = jnp.zeros_like(o_ref)

    def add_one(s, carry):
        i = inv_ref[s]
        t = tgt_ref[i]
        o_ref[pl.ds(t - b * BLOCK, 1), :] += stim_ref[pl.ds(i, 1), :]
        return carry

    jax.lax.fori_loop(starts_ref[b], starts_ref[b + 1], add_one, 0)


def kernel(stimuli, targets):
    tgt = targets.astype(jnp.int32)
    idx = jnp.arange(N_TGT, dtype=jnp.int32)
    lt = tgt[None, :] < tgt[:, None]
    tie = jnp.logical_and(tgt[None, :] == tgt[:, None], idx[None, :] < idx[:, None])
    rank = jnp.sum(jnp.logical_or(lt, tie), axis=1).astype(jnp.int32)
    edges = jnp.arange(N_ROWS // BLOCK + 1, dtype=jnp.int32)[:, None] * BLOCK
    starts = jnp.sum(tgt[None, :] < edges, axis=1).astype(jnp.int32)

    grid_spec = pltpu.PrefetchScalarGridSpec(
        num_scalar_prefetch=3,
        grid=(N_ROWS // BLOCK,),
        in_specs=[
            pl.BlockSpec((N_TGT, T_COLS), lambda b, *_: (0, 0)),
        ],
        out_specs=pl.BlockSpec((BLOCK, T_COLS), lambda b, *_: (b, 0)),
        scratch_shapes=[pltpu.SMEM((N_TGT,), jnp.int32)],
    )
    return pl.pallas_call(
        _body,
        grid_spec=grid_spec,
        out_shape=jax.ShapeDtypeStruct((N_ROWS, T_COLS), jnp.float32),
    )(tgt, rank, starts, stimuli)
